# baseline (speedup 1.0000x reference)
import jax
import jax.numpy as jnp
from jax.experimental import pallas as pl
from jax.experimental.pallas import tpu as pltpu

_N = 16 * 32 * 32
_K = 1024
_C = 64
_BLK = 2048
_PSPLIT = 2
_NBLK = _N // _BLK // _PSPLIT


def _acc_body(code_ref, q_ref, out_ref):
    i = pl.program_id(1)

    @pl.when(i == 0)
    def _init():
        out_ref[...] = jnp.zeros_like(out_ref)

    q_aug = jnp.concatenate(
        [q_ref[...], jnp.ones((_BLK, 1), dtype=jnp.float32)], axis=1)
    out_ref[...] += jax.lax.dot_general(
        code_ref[...],
        q_aug,
        dimension_numbers=(((0,), (0,)), ((), ())),
        preferred_element_type=jnp.float32,
    )[None]


def _loss_body(acc_ref, cb_ref, out_ref):
    acc = acc_ref[0] + acc_ref[1]
    count = acc[:, _C:_C + 1]
    mean_q = acc[:, :_C] / count
    mask = count != 0.0
    sq = (cb_ref[...] - mean_q) ** 2
    sq = jnp.where(mask, sq, 0.0)
    n_selected = jnp.sum(mask.astype(jnp.float32)) * _C
    out_ref[...] = jnp.reshape(jnp.sum(sq) / n_selected, (1, 1))


def kernel(quantized, code, codebook):
    code2d = code.reshape(_N, _K)
    q2d = quantized.reshape(_N, _C)

    partial = pl.pallas_call(
        _acc_body,
        grid=(_PSPLIT, _NBLK),
        in_specs=[
            pl.BlockSpec((_BLK, _K), lambda j, i: (j * _NBLK + i, 0)),
            pl.BlockSpec((_BLK, _C), lambda j, i: (j * _NBLK + i, 0)),
        ],
        out_specs=pl.BlockSpec((1, _K, _C + 1), lambda j, i: (j, 0, 0)),
        out_shape=jax.ShapeDtypeStruct((_PSPLIT, _K, _C + 1), jnp.float32),
        compiler_params=pltpu.CompilerParams(
            dimension_semantics=("parallel", "arbitrary"),
        ),
    )(code2d, q2d)

    out = pl.pallas_call(
        _loss_body,
        out_shape=jax.ShapeDtypeStruct((1, 1), jnp.float32),
    )(partial, codebook)
    return out[0, 0]


# speedup vs baseline: 1.1149x; 1.1149x over previous
import jax
import jax.numpy as jnp
from jax.experimental import pallas as pl
from jax.experimental.pallas import tpu as pltpu

_N = 16 * 32 * 32
_K = 1024
_C = 64
_BLK = 2048
_NBLK = _N // _BLK


def _body(code_ref, q_ref, cb_ref, out_ref, acc_ref):
    i = pl.program_id(0)

    @pl.when(i == 0)
    def _init():
        acc_ref[...] = jnp.zeros_like(acc_ref)

    q_aug = jnp.concatenate(
        [q_ref[...], jnp.ones((_BLK, 1), dtype=jnp.float32)], axis=1)
    acc_ref[...] += jax.lax.dot_general(
        q_aug,
        code_ref[...],
        dimension_numbers=(((0,), (0,)), ((), ())),
        preferred_element_type=jnp.float32,
    )

    @pl.when(i == _NBLK - 1)
    def _epilogue():
        acc = acc_ref[...]
        count = acc[_C:_C + 1, :]
        mean_q = acc[:_C, :] / count
        mask = count != 0.0
        cb_t = jnp.transpose(cb_ref[...])
        sq = (cb_t - mean_q) ** 2
        sq = jnp.where(mask, sq, 0.0)
        n_selected = jnp.sum(mask.astype(jnp.float32)) * _C
        out_ref[...] = jnp.reshape(jnp.sum(sq) / n_selected, (1, 1))


def kernel(quantized, code, codebook):
    code2d = code.reshape(_N, _K)
    q2d = quantized.reshape(_N, _C)

    out = pl.pallas_call(
        _body,
        grid=(_NBLK,),
        in_specs=[
            pl.BlockSpec((_BLK, _K), lambda i: (i, 0)),
            pl.BlockSpec((_BLK, _C), lambda i: (i, 0)),
            pl.BlockSpec((_K, _C), lambda i: (0, 0)),
        ],
        out_specs=pl.BlockSpec((1, 1), lambda i: (0, 0)),
        out_shape=jax.ShapeDtypeStruct((1, 1), jnp.float32),
        scratch_shapes=[pltpu.VMEM((_C + 1, _K), jnp.float32)],
        compiler_params=pltpu.CompilerParams(
            dimension_semantics=("arbitrary",),
        ),
    )(code2d, q2d, codebook)
    return out[0, 0]
